# Initial kernel scaffold; baseline (speedup 1.0000x reference)
import jax
import jax.numpy as jnp
from jax.experimental import pallas as pl


def kernel(edge_index, weight):
    raise NotImplementedError("write your pallas kernel here")


# speedup vs baseline: 4.3075x; 4.3075x over previous
import functools

import jax
import jax.numpy as jnp
from jax import lax
from jax.experimental import pallas as pl
from jax.experimental.pallas import tpu as pltpu
from jax.experimental.pallas import tpu_sc as plsc

E = 2048
NS = 16
L = 16
EDGES_PER_TILE = E // NS
NV = EDGES_PER_TILE // L


def _ncut_body(row_hbm, col_hbm, w_hbm, out_hbm, rows_v, cols_v, w_v,
               acc_v, gat_v, res_v):
    c = lax.axis_index("c")
    s = lax.axis_index("s")

    @pl.when(c == 0)
    def _work():
        base = s * EDGES_PER_TILE
        pltpu.sync_copy(row_hbm.at[pl.ds(base, EDGES_PER_TILE)], rows_v)
        pltpu.sync_copy(col_hbm.at[pl.ds(base, EDGES_PER_TILE)], cols_v)
        pltpu.sync_copy(w_hbm.at[pl.ds(base, EDGES_PER_TILE)], w_v)
        lane = lax.iota(jnp.int32, L)
        zero = jnp.zeros((L,), jnp.float32)
        den = [zero] * 8
        num = [zero] * 8
        for j in range(NV):
            r = rows_v[pl.ds(j * L, L)]
            cl = cols_v[pl.ds(j * L, L)]
            w = w_v[pl.ds(j * L, L)]
            kr = lax.shift_right_logical(r, 3)
            kc = lax.shift_right_logical(cl, 3)
            same = kr == kc
            for k in range(8):
                mr = kr == k
                den[k] = den[k] + jnp.where(mr, w, 0.0) + jnp.where(kc == k, w, 0.0)
                num[k] = num[k] + jnp.where(jnp.logical_and(mr, same), w, 0.0)
        part = zero
        for k in range(8):
            dk = jnp.sum(den[k])
            nk = jnp.sum(num[k])
            part = jnp.where(lane == k, dk, part)
            part = jnp.where(lane == k + 8, nk, part)
        acc_v[...] = part
        pltpu.sync_copy(acc_v, out_hbm.at[pl.ds(s * L, L)])

    plsc.subcore_barrier()

    @pl.when(jnp.logical_and(c == 0, s == 0))
    def _finalize():
        pltpu.sync_copy(out_hbm.at[pl.ds(0, NS * L)], gat_v)
        total = gat_v[pl.ds(0, L)]
        for i in range(1, NS):
            total = total + gat_v[pl.ds(i * L, L)]
        res_v[...] = total
        lane = lax.iota(jnp.int32, L)
        den = total
        num = 2.0 * plsc.load_gather(res_v, [jnp.bitwise_and(lane + 8, 15)])
        ratio = jnp.where(lane < 8, num / den, 0.0)
        out = 1.0 - jnp.sum(ratio) * 0.125
        acc_v[...] = jnp.where(lane == 0, out, 0.0)
        pltpu.sync_copy(acc_v, out_hbm.at[pl.ds(NS * L, L)])


@functools.partial(
    pl.kernel,
    out_type=jax.ShapeDtypeStruct(((NS + 1) * L,), jnp.float32),
    mesh=plsc.VectorSubcoreMesh(core_axis_name="c", subcore_axis_name="s",
                                num_cores=2, num_subcores=16),
    scratch_types=[
        pltpu.VMEM((EDGES_PER_TILE,), jnp.int32),
        pltpu.VMEM((EDGES_PER_TILE,), jnp.int32),
        pltpu.VMEM((EDGES_PER_TILE,), jnp.float32),
        pltpu.VMEM((L,), jnp.float32),
        pltpu.VMEM((NS * L,), jnp.float32),
        pltpu.VMEM((L,), jnp.float32),
    ],
    compiler_params=pltpu.CompilerParams(needs_layout_passes=False),
)
def _ncut_sc(row_hbm, col_hbm, w_hbm, out_hbm, rows_v, cols_v, w_v,
             acc_v, gat_v, res_v):
    _ncut_body(row_hbm, col_hbm, w_hbm, out_hbm, rows_v, cols_v, w_v,
               acc_v, gat_v, res_v)


def kernel(edge_index, weight):
    out = _ncut_sc(edge_index[0], edge_index[1], weight)
    return out[NS * L]


# speedup vs baseline: 4.6205x; 1.0727x over previous
import functools

import jax
import jax.numpy as jnp
from jax import lax
from jax.experimental import pallas as pl
from jax.experimental.pallas import tpu as pltpu
from jax.experimental.pallas import tpu_sc as plsc

E = 2048
NS = 16
L = 16
EDGES_PER_TILE = E // NS
NV = EDGES_PER_TILE // L


def _ncut_body(row_hbm, col_hbm, w_hbm, out_hbm, rows_v, cols_v, w_v,
               acc_v, gat_v, res_v):
    c = lax.axis_index("c")
    s = lax.axis_index("s")

    @pl.when(c == 0)
    def _work():
        base = s * EDGES_PER_TILE
        pltpu.sync_copy(row_hbm.at[pl.ds(base, EDGES_PER_TILE)], rows_v)
        pltpu.sync_copy(col_hbm.at[pl.ds(base, EDGES_PER_TILE)], cols_v)
        pltpu.sync_copy(w_hbm.at[pl.ds(base, EDGES_PER_TILE)], w_v)
        lane = lax.iota(jnp.int32, L)
        zero = jnp.zeros((L,), jnp.float32)
        den = [zero] * 8
        num = [zero] * 8
        for j in range(NV):
            r = rows_v[pl.ds(j * L, L)]
            cl = cols_v[pl.ds(j * L, L)]
            w = w_v[pl.ds(j * L, L)]
            kr = lax.shift_right_logical(r, 3)
            kc = lax.shift_right_logical(cl, 3)
            same = kr == kc
            for k in range(8):
                mr = kr == k
                den[k] = den[k] + jnp.where(mr, w, 0.0) + jnp.where(kc == k, w, 0.0)
                num[k] = num[k] + jnp.where(jnp.logical_and(mr, same), w, 0.0)
        part = zero
        for k in range(8):
            dk = jnp.sum(den[k])
            nk = jnp.sum(num[k])
            part = jnp.where(lane == k, dk, part)
            part = jnp.where(lane == k + 8, nk, part)
        acc_v[...] = part
        pltpu.sync_copy(acc_v, out_hbm.at[pl.ds(s * L, L)])

    plsc.subcore_barrier()

    @pl.when(jnp.logical_and(c == 0, s == 0))
    def _finalize():
        pltpu.sync_copy(out_hbm.at[pl.ds(0, NS * L)], gat_v)
        total = gat_v[pl.ds(0, L)]
        for i in range(1, NS):
            total = total + gat_v[pl.ds(i * L, L)]
        res_v[...] = total
        lane = lax.iota(jnp.int32, L)
        den = total
        num = 2.0 * plsc.load_gather(res_v, [jnp.bitwise_and(lane + 8, 15)])
        ratio = jnp.where(lane < 8, num / den, 0.0)
        out = 1.0 - jnp.sum(ratio) * 0.125
        acc_v[...] = jnp.where(lane == 0, out, 0.0)
        pltpu.sync_copy(acc_v, out_hbm.at[pl.ds(NS * L, L)])


@functools.partial(
    pl.kernel,
    out_type=jax.ShapeDtypeStruct(((NS + 1) * L,), jnp.float32),
    mesh=plsc.VectorSubcoreMesh(core_axis_name="c", subcore_axis_name="s",
                                num_cores=1, num_subcores=16),
    scratch_types=[
        pltpu.VMEM((EDGES_PER_TILE,), jnp.int32),
        pltpu.VMEM((EDGES_PER_TILE,), jnp.int32),
        pltpu.VMEM((EDGES_PER_TILE,), jnp.float32),
        pltpu.VMEM((L,), jnp.float32),
        pltpu.VMEM((NS * L,), jnp.float32),
        pltpu.VMEM((L,), jnp.float32),
    ],
    compiler_params=pltpu.CompilerParams(needs_layout_passes=False),
)
def _ncut_sc(row_hbm, col_hbm, w_hbm, out_hbm, rows_v, cols_v, w_v,
             acc_v, gat_v, res_v):
    _ncut_body(row_hbm, col_hbm, w_hbm, out_hbm, rows_v, cols_v, w_v,
               acc_v, gat_v, res_v)


def kernel(edge_index, weight):
    out = _ncut_sc(edge_index[0], edge_index[1], weight)
    return out[NS * L]


# speedup vs baseline: 4.8702x; 1.0540x over previous
import functools

import jax
import jax.numpy as jnp
from jax import lax
from jax.experimental import pallas as pl
from jax.experimental.pallas import tpu as pltpu
from jax.experimental.pallas import tpu_sc as plsc

E = 2048
NS = 16
L = 16
EDGES_PER_TILE = E // NS
NV = EDGES_PER_TILE // L


def _ncut_body(row_hbm, col_hbm, w_hbm, out_hbm, rows_v, cols_v, w_v,
               acc_v, gat_v, res_v, sem_r, sem_c, sem_w):
    s = lax.axis_index("s")

    base = s * EDGES_PER_TILE
    cp_r = pltpu.async_copy(row_hbm.at[pl.ds(base, EDGES_PER_TILE)], rows_v, sem_r)
    cp_c = pltpu.async_copy(col_hbm.at[pl.ds(base, EDGES_PER_TILE)], cols_v, sem_c)
    cp_w = pltpu.async_copy(w_hbm.at[pl.ds(base, EDGES_PER_TILE)], w_v, sem_w)
    cp_r.wait()
    cp_c.wait()
    cp_w.wait()
    lane = lax.iota(jnp.int32, L)
    zero = jnp.zeros((L,), jnp.float32)

    def step(j, accs):
        off = j * L
        r = rows_v[pl.ds(off, L)]
        cl = cols_v[pl.ds(off, L)]
        w = w_v[pl.ds(off, L)]
        kr = lax.shift_right_logical(r, 3)
        kc = lax.shift_right_logical(cl, 3)
        same = kr == kc
        out = []
        for k in range(8):
            mr = kr == k
            d = accs[k] + jnp.where(mr, w, 0.0) + jnp.where(kc == k, w, 0.0)
            n = accs[8 + k] + jnp.where(jnp.logical_and(mr, same), w, 0.0)
            out.append((d, n))
        return tuple(d for d, _ in out) + tuple(n for _, n in out)

    accs = lax.fori_loop(0, NV, step, (zero,) * 16)
    part = zero
    for k in range(8):
        part = jnp.where(lane == k, jnp.sum(accs[k]), part)
        part = jnp.where(lane == k + 8, jnp.sum(accs[8 + k]), part)
    acc_v[...] = part
    pltpu.sync_copy(acc_v, out_hbm.at[pl.ds(s * L, L)])

    plsc.subcore_barrier()

    @pl.when(s == 0)
    def _finalize():
        pltpu.sync_copy(out_hbm.at[pl.ds(0, NS * L)], gat_v)
        total = gat_v[pl.ds(0, L)]
        for i in range(1, NS):
            total = total + gat_v[pl.ds(i * L, L)]
        res_v[...] = total
        lane = lax.iota(jnp.int32, L)
        den = total
        num = 2.0 * plsc.load_gather(res_v, [jnp.bitwise_and(lane + 8, 15)])
        ratio = jnp.where(lane < 8, num / den, 0.0)
        out = 1.0 - jnp.sum(ratio) * 0.125
        acc_v[...] = jnp.where(lane == 0, out, 0.0)
        pltpu.sync_copy(acc_v, out_hbm.at[pl.ds(NS * L, L)])


@functools.partial(
    pl.kernel,
    out_type=jax.ShapeDtypeStruct(((NS + 1) * L,), jnp.float32),
    mesh=plsc.VectorSubcoreMesh(core_axis_name="c", subcore_axis_name="s",
                                num_cores=1, num_subcores=16),
    scratch_types=[
        pltpu.VMEM((EDGES_PER_TILE,), jnp.int32),
        pltpu.VMEM((EDGES_PER_TILE,), jnp.int32),
        pltpu.VMEM((EDGES_PER_TILE,), jnp.float32),
        pltpu.VMEM((L,), jnp.float32),
        pltpu.VMEM((NS * L,), jnp.float32),
        pltpu.VMEM((L,), jnp.float32),
        pltpu.SemaphoreType.DMA,
        pltpu.SemaphoreType.DMA,
        pltpu.SemaphoreType.DMA,
    ],
    compiler_params=pltpu.CompilerParams(needs_layout_passes=False),
)
def _ncut_sc(row_hbm, col_hbm, w_hbm, out_hbm, rows_v, cols_v, w_v,
             acc_v, gat_v, res_v, sem_r, sem_c, sem_w):
    _ncut_body(row_hbm, col_hbm, w_hbm, out_hbm, rows_v, cols_v, w_v,
               acc_v, gat_v, res_v, sem_r, sem_c, sem_w)


def kernel(edge_index, weight):
    out = _ncut_sc(edge_index[0], edge_index[1], weight)
    return out[NS * L]
